# Initial kernel scaffold; baseline (speedup 1.0000x reference)
import functools

import jax
import jax.numpy as jnp
from jax import lax
from jax.experimental import pallas as pl
from jax.experimental.pallas import tpu as pltpu
from jax.experimental.pallas import tpu_sc as plsc

N_PIECES = 16
N_FEATURES = 1024
BATCH = 8192
LOWER_X, UPPER_X = 0.0, 1.0
LOWER_Y, UPPER_Y = 0.0, 1.0

NW = 32
FB = 16
NBLK = N_FEATURES // FB
BLK_PER_W = NBLK // NW
R = 1024
NCHUNK = BATCH // R

TAB_ROWS = 3 * N_PIECES + 1


def _prep_body(xx_ref, xdx_ref, yx_ref, ydx_ref, tab_ref):
    def piece_rows(x0, dx_ref, lower, upper):
        cums = []
        acc = None
        for i in range(N_PIECES):
            v = dx_ref[i]
            sp = jnp.maximum(v, 0.0) + jnp.log(1.0 + jnp.exp(-jnp.abs(v)))
            acc = sp if acc is None else acc + sp
            cums.append(acc)
        xc = [x0 - lower] + [(x0 + d) - lower for d in cums]
        change = (upper - lower) / (xc[-1] - xc[0])
        return [c * change + lower for c in xc]

    xp = piece_rows(xx_ref[0], xdx_ref, LOWER_X, UPPER_X)
    yp = piece_rows(yx_ref[0], ydx_ref, LOWER_Y, UPPER_Y)
    for i in range(N_PIECES + 1):
        tab_ref[i, :] = xp[i]
    for i in range(N_PIECES):
        s = (yp[i + 1] - yp[i]) / (xp[i + 1] - xp[i])
        tab_ref[N_PIECES + 1 + i, :] = s
        tab_ref[2 * N_PIECES + 1 + i, :] = yp[i] - xp[i] * s


_prep = pl.pallas_call(
    _prep_body,
    out_shape=jax.ShapeDtypeStruct((TAB_ROWS, N_FEATURES), jnp.float32),
)


def _sc_body(x_hbm, tab_hbm, out_hbm, tab_v, in_v, out_v):
    cid = lax.axis_index("c")
    sid = lax.axis_index("s")
    wid = sid * 2 + cid
    lane = lax.iota(jnp.int32, 16)
    for bi in range(BLK_PER_W):
        f0 = (wid * BLK_PER_W + bi) * FB
        pltpu.sync_copy(tab_hbm.at[:, pl.ds(f0, FB)], tab_v)
        xp = [tab_v[i] for i in range(N_PIECES + 1)]

        def chunk(ci, _):
            r0 = ci * R
            pltpu.sync_copy(x_hbm.at[pl.ds(r0, R), pl.ds(f0, FB)], in_v)

            def row(r, _):
                xv = in_v[r]
                seg = jnp.zeros((16,), jnp.int32)
                for i in range(1, N_PIECES):
                    seg = seg + (xv >= xp[i]).astype(jnp.int32)
                sl = plsc.load_gather(tab_v, [seg + (N_PIECES + 1), lane])
                cc = plsc.load_gather(tab_v, [seg + (2 * N_PIECES + 1), lane])
                val = xv * sl + cc
                inb = (xv >= xp[0]) & (xv <= xp[N_PIECES])
                out_v[r] = jnp.where(inb, val, xv)
                return 0

            lax.fori_loop(0, R, row, 0)
            pltpu.sync_copy(out_v, out_hbm.at[pl.ds(r0, R), pl.ds(f0, FB)])
            return 0

        lax.fori_loop(0, NCHUNK, chunk, 0)


_sc_main = functools.partial(
    pl.kernel,
    mesh=plsc.VectorSubcoreMesh(core_axis_name="c", subcore_axis_name="s"),
    out_type=jax.ShapeDtypeStruct((BATCH, N_FEATURES), jnp.float32),
    scratch_types=[
        pltpu.VMEM((TAB_ROWS, FB), jnp.float32),
        pltpu.VMEM((R, FB), jnp.float32),
        pltpu.VMEM((R, FB), jnp.float32),
    ],
)(_sc_body)


def kernel(x, xr_x, xr_dx, yr_x, yr_dx):
    xx = xr_x[0].reshape(1, N_FEATURES)
    yx = yr_x[0].reshape(1, N_FEATURES)
    xdxT = xr_dx[0].T
    ydxT = yr_dx[0].T
    tab = _prep(xx, xdxT, yx, ydxT)
    return _sc_main(x, tab)


# speedup vs baseline: 1.4119x; 1.4119x over previous
import functools

import jax
import jax.numpy as jnp
from jax import lax
from jax.experimental import pallas as pl
from jax.experimental.pallas import tpu as pltpu
from jax.experimental.pallas import tpu_sc as plsc

N_PIECES = 16
N_FEATURES = 1024
BATCH = 8192
LOWER_X, UPPER_X = 0.0, 1.0
LOWER_Y, UPPER_Y = 0.0, 1.0

NW = 32
CR = 16
NCHUNK = BATCH // CR
CHUNKS_PER_W = NCHUNK // NW
NGROUP = N_FEATURES // 16

TAB_ROWS = 56


def _prep_body(xx_ref, xdx_ref, yx_ref, ydx_ref, tab_ref):
    def piece_rows(x0, dx_ref, lower, upper):
        cums = []
        acc = None
        for i in range(N_PIECES):
            v = dx_ref[i]
            sp = jnp.maximum(v, 0.0) + jnp.log(1.0 + jnp.exp(-jnp.abs(v)))
            acc = sp if acc is None else acc + sp
            cums.append(acc)
        xc = [x0 - lower] + [(x0 + d) - lower for d in cums]
        change = (upper - lower) / (xc[-1] - xc[0])
        return [c * change + lower for c in xc]

    xp = piece_rows(xx_ref[0], xdx_ref, LOWER_X, UPPER_X)
    yp = piece_rows(yx_ref[0], ydx_ref, LOWER_Y, UPPER_Y)
    for i in range(N_PIECES + 1):
        tab_ref[i, :] = xp[i]
    for i in range(N_PIECES):
        s = (yp[i + 1] - yp[i]) / (xp[i + 1] - xp[i])
        tab_ref[N_PIECES + 1 + i, :] = s
        tab_ref[2 * N_PIECES + 1 + i, :] = yp[i] - xp[i] * s
    for i in range(3 * N_PIECES + 1, TAB_ROWS):
        tab_ref[i, :] = jnp.zeros((N_FEATURES,), jnp.float32)


_prep = pl.pallas_call(
    _prep_body,
    out_shape=jax.ShapeDtypeStruct((TAB_ROWS, N_FEATURES), jnp.float32),
)


def _sc_body(x_hbm, tab_hbm, out_hbm, tab_v, in_v, out_v):
    cid = lax.axis_index("c")
    sid = lax.axis_index("s")
    wid = sid * 2 + cid
    lane = lax.iota(jnp.int32, 16)
    pltpu.sync_copy(tab_hbm, tab_v)

    def chunk_body(ci, _):
        chunk = wid * CHUNKS_PER_W + ci
        pltpu.sync_copy(x_hbm.at[chunk], in_v)
        def kgroup(k, _):
            c0 = k * 16
            xp = [tab_v[i, pl.ds(c0, 16)] for i in range(N_PIECES + 1)]
            col = c0 + lane
            def row(r, _):
                xv = in_v[r, pl.ds(c0, 16)]
                seg = jnp.zeros((16,), jnp.int32)
                for i in range(1, N_PIECES):
                    seg = seg + jnp.where(xv >= xp[i], 1, 0)
                sl = plsc.load_gather(tab_v, [seg + (N_PIECES + 1), col])
                cc = plsc.load_gather(tab_v, [seg + (2 * N_PIECES + 1), col])
                val = xv * sl + cc
                inb = (xv >= xp[0]) & (xv <= xp[N_PIECES])
                out_v[r, pl.ds(c0, 16)] = jnp.where(inb, val, xv)
                return 0
            lax.fori_loop(0, CR, row, 0)
            return 0
        lax.fori_loop(0, NGROUP, kgroup, 0)
        pltpu.sync_copy(out_v, out_hbm.at[chunk])
        return 0

    lax.fori_loop(0, CHUNKS_PER_W, chunk_body, 0)


_sc_main = functools.partial(
    pl.kernel,
    mesh=plsc.VectorSubcoreMesh(core_axis_name="c", subcore_axis_name="s"),
    compiler_params=pltpu.CompilerParams(needs_layout_passes=False),
    out_type=jax.ShapeDtypeStruct((NCHUNK, CR, N_FEATURES), jnp.float32),
    scratch_types=[
        pltpu.VMEM((TAB_ROWS, N_FEATURES), jnp.float32),
        pltpu.VMEM((CR, N_FEATURES), jnp.float32),
        pltpu.VMEM((CR, N_FEATURES), jnp.float32),
    ],
)(_sc_body)


def kernel(x, xr_x, xr_dx, yr_x, yr_dx):
    xx = xr_x[0].reshape(1, N_FEATURES)
    yx = yr_x[0].reshape(1, N_FEATURES)
    xdxT = xr_dx[0].T
    ydxT = yr_dx[0].T
    tab = _prep(xx, xdxT, yx, ydxT)
    out = _sc_main(x.reshape(NCHUNK, CR, N_FEATURES), tab)
    return out.reshape(BATCH, N_FEATURES)
